# baseline (speedup 1.0000x reference)
import functools

import jax
import jax.numpy as jnp
from jax.experimental import pallas as pl
from jax.experimental.pallas import tpu as pltpu

SENT = 64
ANN = 256
FC7 = 2048
JEMB = 512
SB = 8

HI = jax.lax.Precision.HIGHEST


def _kernel(attn, cxt_feats, dist2, fc_w, w7, lw, b2, cw,
            fuse, dists, maxid, base_s, cf_s, w2_s):
    pid = pl.program_id(0)

    @pl.when(pid == 0)
    def _prologue():
        a = attn[...]
        m = jnp.max(a, axis=1, keepdims=True)
        cols = jax.lax.broadcasted_iota(jnp.int32, (SENT, ANN), 1)
        ids = jnp.min(jnp.where(a == m, cols, ANN), axis=1,
                      keepdims=True)
        maxid[...] = ids
        onehot = (cols == ids).astype(jnp.float32)
        ok = jnp.where(m == 0.0, 0.0, 1.0)

        dg = jax.lax.dot_general(onehot, dist2[...],
                                 (((1,), (1,)), ((), ())), precision=HI)
        dists[...] = jnp.where(ok == 0.0, 100.0, dg)

        w2_s[...] = jnp.transpose(fc_w[:, FC7:FC7 + 5] * lw[...]) \
            .reshape(5, JEMB)

        g = jax.lax.dot(onehot, cw[...], precision=HI)
        ss = g[:, :ANN] * g[:, :ANN]
        for c in range(1, 5):
            lc = g[:, c * ANN:(c + 1) * ANN]
            ss = ss + lc * lc
        invl = ok / jnp.maximum(jnp.sqrt(ss), 1e-12)
        cf_s[...] = g * jnp.concatenate([invl] * 5, axis=1)

        rf = jax.lax.dot(onehot, cxt_feats[...], precision=HI)
        n = jnp.sqrt(jnp.sum(rf * rf, axis=1, keepdims=True))
        inv7 = ok / jnp.maximum(n, 1e-12)
        rfn = rf * inv7 * w7[...]
        base_s[...] = jax.lax.dot_general(rfn, fc_w[:, :FC7],
                                          (((1,), (1,)), ((), ()))) + b2[...]

    sl = pl.ds(pid * SB, SB)
    cf = cf_s[sl, :]
    w2 = w2_s[...]
    baseb = base_s[sl, :][:, None, :]
    CH = 4
    for j in range(0, ANN, CH):
        acc = jnp.broadcast_to(baseb, (SB, CH, JEMB))
        for c in range(5):
            acc = acc + cf[:, c * ANN + j:c * ANN + j + CH][:, :, None] \
                * w2[c][None, None, :]
        fuse[:, j:j + CH, :] = acc


@functools.partial(jax.jit, static_argnames=("interpret",))
def _run(cxt_feats, cxt_lfeats, obj_attn, dist, fc7_norm_w, lfeat_norm_w,
         fc_w, fc_b, interpret=False):
    cw = jnp.transpose(cxt_lfeats, (1, 2, 0)).reshape(ANN, 5 * ANN)
    dist2 = dist.reshape(ANN, ANN)
    b2 = fc_b.reshape(1, JEMB)

    fuse, dists, maxid = pl.pallas_call(
        _kernel,
        grid=(SENT // SB,),
        in_specs=[
            pl.BlockSpec((SENT, ANN), lambda i: (0, 0)),
            pl.BlockSpec((ANN, FC7), lambda i: (0, 0)),
            pl.BlockSpec((ANN, ANN), lambda i: (0, 0)),
            pl.BlockSpec((JEMB, FC7 + 5), lambda i: (0, 0)),
            pl.BlockSpec((1, FC7), lambda i: (0, 0)),
            pl.BlockSpec((1, 5), lambda i: (0, 0)),
            pl.BlockSpec((1, JEMB), lambda i: (0, 0)),
            pl.BlockSpec((ANN, 5 * ANN), lambda i: (0, 0)),
        ],
        out_specs=[
            pl.BlockSpec((SB, ANN, JEMB), lambda i: (i, 0, 0)),
            pl.BlockSpec((SENT, ANN), lambda i: (0, 0)),
            pl.BlockSpec((SENT, 1), lambda i: (0, 0)),
        ],
        out_shape=[
            jax.ShapeDtypeStruct((SENT, ANN, JEMB), jnp.float32),
            jax.ShapeDtypeStruct((SENT, ANN), jnp.float32),
            jax.ShapeDtypeStruct((SENT, 1), jnp.int32),
        ],
        scratch_shapes=[
            pltpu.VMEM((SENT, JEMB), jnp.float32),
            pltpu.VMEM((SENT, 5 * ANN), jnp.float32),
            pltpu.VMEM((5, JEMB), jnp.float32),
        ],
        interpret=interpret,
    )(obj_attn, cxt_feats, dist2, fc_w, fc7_norm_w, lfeat_norm_w, b2, cw)

    return fuse, dists, maxid[:, 0]


def kernel(cxt_feats, cxt_lfeats, obj_attn, wo_obj_idx, dist,
           fc7_norm_w, lfeat_norm_w, fc_w, fc_b):
    del wo_obj_idx
    return _run(cxt_feats, cxt_lfeats, obj_attn, dist, fc7_norm_w,
                lfeat_norm_w, fc_w, fc_b)


# speedup vs baseline: 1.3688x; 1.3688x over previous
import functools

import jax
import jax.numpy as jnp
from jax.experimental import pallas as pl
from jax.experimental.pallas import tpu as pltpu

SENT = 64
ANN = 256
FC7 = 2048
JEMB = 512
SB = 8

HI = jax.lax.Precision.HIGHEST


def _kernel(attn, cxt_feats, dist2, fc_w, w7, lw, b2, cw,
            fuse, dists, maxid, base_s, cf_s, w2_s):
    pid = pl.program_id(0)

    @pl.when(pid == 0)
    def _prologue():
        a = attn[...]
        m = jnp.max(a, axis=1, keepdims=True)
        cols = jax.lax.broadcasted_iota(jnp.int32, (SENT, ANN), 1)
        ids = jnp.min(jnp.where(a == m, cols, ANN), axis=1,
                      keepdims=True)
        maxid[...] = ids
        onehot = (cols == ids).astype(jnp.float32)
        ok = jnp.where(m == 0.0, 0.0, 1.0)

        dg = jax.lax.dot_general(onehot, dist2[...],
                                 (((1,), (1,)), ((), ())), precision=HI)
        dists[...] = jnp.where(ok == 0.0, 100.0, dg)

        w2_s[...] = jnp.transpose(fc_w[:, FC7:FC7 + 5] * lw[...]) \
            .reshape(5, JEMB)

        g = jax.lax.dot(onehot, cw[...], precision=HI)
        ss = g[:, :ANN] * g[:, :ANN]
        for c in range(1, 5):
            lc = g[:, c * ANN:(c + 1) * ANN]
            ss = ss + lc * lc
        invl = ok / jnp.maximum(jnp.sqrt(ss), 1e-12)
        cf_s[...] = g * jnp.concatenate([invl] * 5, axis=1)

        rf = jax.lax.dot(onehot, cxt_feats[...], precision=HI)
        n = jnp.sqrt(jnp.sum(rf * rf, axis=1, keepdims=True))
        inv7 = ok / jnp.maximum(n, 1e-12)
        rfn = rf * inv7 * w7[...]
        base_s[...] = jax.lax.dot_general(rfn, fc_w[:, :FC7],
                                          (((1,), (1,)), ((), ()))) + b2[...]

    sl = pl.ds(pid * SB, SB)
    cf = cf_s[sl, :]
    w2 = w2_s[...]
    baseb = base_s[sl, :][:, None, :]
    CH = 8
    for j in range(0, ANN, CH):
        acc = jnp.broadcast_to(baseb, (SB, CH, JEMB))
        for c in range(5):
            acc = acc + cf[:, c * ANN + j:c * ANN + j + CH][:, :, None] \
                * w2[c][None, None, :]
        fuse[:, j:j + CH, :] = acc


@functools.partial(jax.jit, static_argnames=("interpret",))
def _run(cxt_feats, cxt_lfeats, obj_attn, dist, fc7_norm_w, lfeat_norm_w,
         fc_w, fc_b, interpret=False):
    cw = jnp.transpose(cxt_lfeats, (1, 2, 0)).reshape(ANN, 5 * ANN)
    dist2 = dist.reshape(ANN, ANN)
    b2 = fc_b.reshape(1, JEMB)

    fuse, dists, maxid = pl.pallas_call(
        _kernel,
        grid=(SENT // SB,),
        in_specs=[
            pl.BlockSpec((SENT, ANN), lambda i: (0, 0)),
            pl.BlockSpec((ANN, FC7), lambda i: (0, 0)),
            pl.BlockSpec((ANN, ANN), lambda i: (0, 0)),
            pl.BlockSpec((JEMB, FC7 + 5), lambda i: (0, 0)),
            pl.BlockSpec((1, FC7), lambda i: (0, 0)),
            pl.BlockSpec((1, 5), lambda i: (0, 0)),
            pl.BlockSpec((1, JEMB), lambda i: (0, 0)),
            pl.BlockSpec((ANN, 5 * ANN), lambda i: (0, 0)),
        ],
        out_specs=[
            pl.BlockSpec((SB, ANN, JEMB), lambda i: (i, 0, 0)),
            pl.BlockSpec((SENT, ANN), lambda i: (0, 0)),
            pl.BlockSpec((SENT, 1), lambda i: (0, 0)),
        ],
        out_shape=[
            jax.ShapeDtypeStruct((SENT, ANN, JEMB), jnp.float32),
            jax.ShapeDtypeStruct((SENT, ANN), jnp.float32),
            jax.ShapeDtypeStruct((SENT, 1), jnp.int32),
        ],
        scratch_shapes=[
            pltpu.VMEM((SENT, JEMB), jnp.float32),
            pltpu.VMEM((SENT, 5 * ANN), jnp.float32),
            pltpu.VMEM((5, JEMB), jnp.float32),
        ],
        interpret=interpret,
    )(obj_attn, cxt_feats, dist2, fc_w, fc7_norm_w, lfeat_norm_w, b2, cw)

    return fuse, dists, maxid[:, 0]


def kernel(cxt_feats, cxt_lfeats, obj_attn, wo_obj_idx, dist,
           fc7_norm_w, lfeat_norm_w, fc_w, fc_b):
    del wo_obj_idx
    return _run(cxt_feats, cxt_lfeats, obj_attn, dist, fc7_norm_w,
                lfeat_norm_w, fc_w, fc_b)
